# baseline (speedup 1.0000x reference)
import functools

import jax
import jax.numpy as jnp
from jax import lax
from jax.experimental import pallas as pl
from jax.experimental.pallas import tpu as pltpu
from jax.experimental.pallas import tpu_sc as plsc

N = 10000
E = 320000
D = 128
P = 5
HD = D // 2
NS = 16
L = 16

N_PAD = 10240
RPT = N_PAD // NS
RQ = RPT // 128

CPT = 160
CHW = 256
CPT2 = 80
EPT = CPT * 128
E_PAD = NS * EPT
GRP = 4
NG = CPT // GRP

_f32 = jnp.float32
_i32 = jnp.int32


def _body(x_hbm, g_hbm, src_hbm, dst_hbm, dist_hbm, w_hbm, out_hbm,
          src_v, dst_v, dtmp_v, rows_v, trav_v, g_v, w_v,
          u_hbm, acc_sh, base_hbm, travel_sh, semG, semS):
  c = lax.axis_index("c")
  s = lax.axis_index("s")
  row0 = s * RPT
  col0 = c * HD
  u_sh = u_hbm.at[c]

  pltpu.sync_copy(src_hbm.at[s], src_v)
  pltpu.sync_copy(dst_hbm.at[s], dst_v)
  pltpu.sync_copy(w_hbm, w_v)
  pltpu.sync_copy(g_hbm.at[pl.ds(col0, HD)], g_v)
  wv = w_v[...]
  w1s, w2s, w3s, w4s, w5s = wv[0], wv[1], wv[2], wv[3], wv[4]
  g4 = [g_v[pl.ds(cc * L, L)] for cc in range(HD // L)]

  @pl.loop(0, 128 // L)
  def _(i):
    trav_v[pl.ds(i * L, L)] = jnp.zeros((L,), _f32)

  for q in range(RQ):
    pltpu.sync_copy(trav_v, travel_sh.at[pl.ds(row0 + 128 * q, 128)])
  plsc.subcore_barrier()

  @pl.loop(0, CPT2 // GRP)
  def _(g):
    j0 = g * GRP
    gets = [
        pltpu.async_copy(dist_hbm.at[s, j0 + b], dtmp_v.at[b], semG)
        for b in range(GRP)
    ]
    for h in gets:
      h.wait()
    puts = [
        pltpu.async_copy(dtmp_v.at[b], travel_sh.at[dst_v.at[j0 + b]],
                         semS, add=True)
        for b in range(GRP)
    ]
    for h in puts:
      h.wait()
  plsc.subcore_barrier()

  for q in range(RQ):
    r0 = row0 + 128 * q
    x_v = rows_v.at[0, pl.ds(0, 128)]
    b_v = rows_v.at[1, pl.ds(0, 128)]
    pltpu.sync_copy(x_hbm.at[pl.ds(r0, 128), pl.ds(col0, HD)], x_v)
    pltpu.sync_copy(travel_sh.at[pl.ds(r0, 128)], trav_v)

    @pl.loop(0, 128 // L)
    def _(rg):
      tv16 = trav_v[pl.ds(rg * L, L)]
      tvw16 = w4s * jnp.maximum(w5s * tv16, 0.0)
      for i in range(L):
        r = rg * L + i
        tvw = tvw16[i]
        for cc in range(HD // L):
          sl = pl.ds(cc * L, L)
          b_v[r, sl] = w1s * x_v[r, sl] + w2s * g4[cc] + tvw

    pltpu.sync_copy(b_v, base_hbm.at[c, pl.ds(r0, 128)])
    pltpu.sync_copy(x_v, u_sh.at[pl.ds(r0, 128)])

  plsc.subcore_barrier()

  for k in range(P):
    z_v = rows_v.at[0, pl.ds(128, 128)]

    @pl.loop(0, 128)
    def _(r):
      for cc in range(HD // L):
        z_v[r, pl.ds(cc * L, L)] = jnp.zeros((L,), _f32)

    for q in range(RQ):
      pltpu.sync_copy(z_v, acc_sh.at[pl.ds(row0 + 128 * q, 128)])
    plsc.subcore_barrier()

    NSC = CPT2

    def fire_g(j, b):
      pltpu.async_copy(u_sh.at[src_v.at[j]], rows_v.at[b], semG)

    def wait_g(j, b):
      pltpu.make_async_copy(u_sh.at[src_v.at[j]], rows_v.at[b], semG).wait()

    def fire_s(j, b):
      pltpu.async_copy(rows_v.at[b], acc_sh.at[dst_v.at[j]], semS, add=True)

    def wait_s(j, b):
      pltpu.make_async_copy(rows_v.at[b], acc_sh.at[dst_v.at[j]],
                            semS).wait()

    fire_g(0, 0)

    @pl.loop(0, NSC // 2 - 1)
    def _(t):
      for o in range(2):
        j = 2 * t + o
        b = o
        bn = 1 - o
        wait_g(j, b)
        fire_g(j + 1, bn)
        fire_s(j, b)
        wait_s(j, b)

    for o in range(2):
      j = NSC - 2 + o
      wait_g(j, o)
      if o == 0:
        fire_g(j + 1, 1)
      fire_s(j, o)
      wait_s(j, o)

    plsc.subcore_barrier()

    for q in range(RQ):
      r0 = row0 + 128 * q
      a_v = rows_v.at[0, pl.ds(0, 128)]
      b_v = rows_v.at[1, pl.ds(0, 128)]
      pltpu.sync_copy(acc_sh.at[pl.ds(r0, 128)], a_v)
      pltpu.sync_copy(base_hbm.at[c, pl.ds(r0, 128)], b_v)

      @pl.loop(0, 128)
      def _(r):
        for cc in range(HD // L):
          sl = pl.ds(cc * L, L)
          val = b_v[r, sl] + w3s * a_v[r, sl]
          if k == P - 1:
            val = jnp.maximum(val, 0.0)
          a_v[r, sl] = val

      if k == P - 1:
        pltpu.sync_copy(a_v, out_hbm.at[pl.ds(r0, 128), pl.ds(col0, HD)])
      else:
        pltpu.sync_copy(a_v, u_sh.at[pl.ds(r0, 128)])

    if k != P - 1:
      plsc.subcore_barrier()


@jax.jit
def _run(x_pad, g, src_p, dst_p, dist_p, w_vec):
  mesh = plsc.VectorSubcoreMesh(core_axis_name="c", subcore_axis_name="s")
  f = pl.kernel(
      _body,
      out_type=jax.ShapeDtypeStruct((N_PAD, D), _f32),
      mesh=mesh,
      compiler_params=pltpu.CompilerParams(use_tc_tiling_on_sc=False),
      scratch_types=[
          pltpu.VMEM((CPT2, CHW), _i32),
          pltpu.VMEM((CPT2, CHW), _i32),
          pltpu.VMEM((GRP, CHW), _f32),
          pltpu.VMEM((2, CHW, HD), _f32),
          pltpu.VMEM((128,), _f32),
          pltpu.VMEM((HD,), _f32),
          pltpu.VMEM((16,), _f32),
          pltpu.HBM((2, N_PAD, HD), _f32),
          pltpu.VMEM_SHARED((N_PAD, HD), _f32),
          pltpu.HBM((2, N_PAD, HD), _f32),
          pltpu.VMEM_SHARED((N_PAD,), _f32),
          pltpu.SemaphoreType.DMA,
          pltpu.SemaphoreType.DMA,
      ],
  )
  return f(x_pad, g, src_p, dst_p, dist_p, w_vec)


def kernel(x_full, edge_index, edge_dist, w1, w2, w3, w4, w5):
  x = x_full[:N]
  g = x_full[N]
  x_pad = jnp.zeros((N_PAD, D), _f32).at[:N].set(x)
  pad = E_PAD - E
  src_p = jnp.concatenate(
      [edge_index[0], jnp.zeros((pad,), _i32)]).reshape(NS, CPT2, CHW)
  dst_p = jnp.concatenate(
      [edge_index[1], jnp.full((pad,), N, _i32)]).reshape(NS, CPT2, CHW)
  dist_p = jnp.concatenate(
      [edge_dist, jnp.zeros((pad,), _f32)]).reshape(NS, CPT2, CHW)
  w_vec = jnp.stack([w1, w2, w3, w4, w5] + [jnp.float32(0.0)] * 11)
  out = _run(x_pad, g, src_p, dst_p, dist_p, w_vec)
  return out[:N]


# speedup vs baseline: 1.0846x; 1.0846x over previous
import jax
import jax.numpy as jnp
from jax import lax
from jax.experimental import pallas as pl
from jax.experimental.pallas import tpu as pltpu
from jax.experimental.pallas import tpu_sc as plsc

N = 10000
E = 320000
D = 128
P = 5
HD = D // 2
NS = 16
L = 16

N_PAD = 10240
RPT = N_PAD // NS
RQ = RPT // 128

CPT = 160
EPT = CPT * 128
E_PAD = NS * EPT
GRP = 4
NB = 8

_f32 = jnp.float32
_i32 = jnp.int32


def _body(x_hbm, g_hbm, ei_hbm, dist_hbm, w_hbm, out_hbm,
          eix_v, dtmp_v, rows_v, trav_v, g_v, w_v,
          u_hbm, acc_sh, base_hbm, travel_sh, semG, semS, semI):
  c = lax.axis_index("c")
  s = lax.axis_index("s")
  row0 = s * RPT
  col0 = c * HD
  u_sh = u_hbm.at[c]

  pltpu.sync_copy(w_hbm, w_v)
  pltpu.sync_copy(g_hbm.at[pl.ds(col0, HD)], g_v)
  wv = w_v[...]
  w1s, w2s, w3s, w4s, w5s = wv[0], wv[1], wv[2], wv[3], wv[4]
  g4 = [g_v[pl.ds(cc * L, L)] for cc in range(HD // L)]

  @pl.loop(0, 128 // L)
  def _(i):
    trav_v[pl.ds(i * L, L)] = jnp.zeros((L,), _f32)

  for q in range(RQ):
    pltpu.sync_copy(trav_v, travel_sh.at[pl.ds(row0 + 128 * q, 128)])
  plsc.subcore_barrier()

  @pl.loop(0, CPT // GRP)
  def _(g):
    j0 = g * GRP
    gets = [
        pltpu.async_copy(dist_hbm.at[s, j0 + b], dtmp_v.at[b], semG)
        for b in range(GRP)
    ] + [
        pltpu.async_copy(ei_hbm.at[s, j0 + b], eix_v.at[b], semI)
        for b in range(GRP)
    ]
    for h in gets:
      h.wait()
    puts = [
        pltpu.async_copy(dtmp_v.at[b], travel_sh.at[eix_v.at[b, 1]],
                         semS, add=True)
        for b in range(GRP)
    ]
    for h in puts:
      h.wait()
  plsc.subcore_barrier()

  for q in range(RQ):
    r0 = row0 + 128 * q
    x_v = rows_v.at[0]
    b_v = rows_v.at[1]
    pltpu.sync_copy(x_hbm.at[pl.ds(r0, 128), pl.ds(col0, HD)], x_v)
    pltpu.sync_copy(travel_sh.at[pl.ds(r0, 128)], trav_v)

    @pl.loop(0, 128 // L)
    def _(rg):
      tv16 = trav_v[pl.ds(rg * L, L)]
      tvw16 = w4s * jnp.maximum(w5s * tv16, 0.0)
      for i in range(L):
        r = rg * L + i
        tvw = tvw16[i]
        for cc in range(HD // L):
          sl = pl.ds(cc * L, L)
          b_v[r, sl] = w1s * x_v[r, sl] + w2s * g4[cc] + tvw

    pltpu.sync_copy(b_v, base_hbm.at[c, pl.ds(r0, 128)])
    pltpu.sync_copy(x_v, u_sh.at[pl.ds(r0, 128)])

  plsc.subcore_barrier()

  NI = 12

  def fire_i(j):
    pltpu.async_copy(ei_hbm.at[s, j], eix_v.at[j % NI], semI)

  def wait_i(j):
    pltpu.make_async_copy(ei_hbm.at[s, j], eix_v.at[j % NI], semI).wait()

  def fire_g(j):
    pltpu.async_copy(u_sh.at[eix_v.at[j % NI, 0]], rows_v.at[j % NB], semG)

  def wait_g(j):
    pltpu.make_async_copy(u_sh.at[eix_v.at[j % NI, 0]], rows_v.at[j % NB],
                          semG).wait()

  def fire_s(j):
    pltpu.async_copy(rows_v.at[j % NB], acc_sh.at[eix_v.at[j % NI, 1]],
                     semS, add=True)

  def wait_s(j):
    pltpu.make_async_copy(rows_v.at[j % NB], acc_sh.at[eix_v.at[j % NI, 1]],
                          semS).wait()

  def one_round(is_last):
    z_v = rows_v.at[2]

    @pl.loop(0, 128)
    def _(r):
      for cc in range(HD // L):
        z_v[r, pl.ds(cc * L, L)] = jnp.zeros((L,), _f32)

    for q in range(RQ):
      pltpu.sync_copy(z_v, acc_sh.at[pl.ds(row0 + 128 * q, 128)])
    plsc.subcore_barrier()

    for j in range(8):
      fire_i(j)
    for j in range(4):
      wait_i(j)
      fire_g(j)
    for j in range(4):
      wait_g(j)
      fire_s(j)
      fire_i(j + 8)
      wait_i(j + 4)
      fire_g(j + 4)

    def steady(j):
      wait_g(j)
      fire_s(j)
      wait_s(j - 4)
      fire_i(j + 8)
      wait_i(j + 4)
      fire_g(j + 4)

    @pl.loop(4, 152)
    def _(j):
      steady(j)

    for j in range(152, 156):
      wait_g(j)
      fire_s(j)
      wait_s(j - 4)
      wait_i(j + 4)
      fire_g(j + 4)
    for j in range(156, 160):
      wait_g(j)
      fire_s(j)
      wait_s(j - 4)
    for j in range(156, 160):
      wait_s(j)

    plsc.subcore_barrier()

    for q in range(RQ):
      r0 = row0 + 128 * q
      a_v = rows_v.at[0]
      b_v = rows_v.at[1]
      pltpu.sync_copy(acc_sh.at[pl.ds(r0, 128)], a_v)
      pltpu.sync_copy(base_hbm.at[c, pl.ds(r0, 128)], b_v)

      @pl.loop(0, 128)
      def _(r):
        for cc in range(HD // L):
          sl = pl.ds(cc * L, L)
          val = b_v[r, sl] + w3s * a_v[r, sl]
          if is_last:
            val = jnp.maximum(val, 0.0)
          a_v[r, sl] = val

      if is_last:
        pltpu.sync_copy(a_v, out_hbm.at[pl.ds(r0, 128), pl.ds(col0, HD)])
      else:
        pltpu.sync_copy(a_v, u_sh.at[pl.ds(r0, 128)])

    if not is_last:
      plsc.subcore_barrier()

  @pl.loop(0, P - 1)
  def _(k):
    one_round(False)

  one_round(True)


@jax.jit
def _run(x_pad, g, ei_p, dist_p, w_vec):
  mesh = plsc.VectorSubcoreMesh(core_axis_name="c", subcore_axis_name="s")
  f = pl.kernel(
      _body,
      out_type=jax.ShapeDtypeStruct((N_PAD, D), _f32),
      mesh=mesh,
      compiler_params=pltpu.CompilerParams(use_tc_tiling_on_sc=False),
      scratch_types=[
          pltpu.VMEM((12, 2, 128), _i32),
          pltpu.VMEM((GRP, 128), _f32),
          pltpu.VMEM((NB, 128, HD), _f32),
          pltpu.VMEM((128,), _f32),
          pltpu.VMEM((HD,), _f32),
          pltpu.VMEM((16,), _f32),
          pltpu.HBM((2, N_PAD, HD), _f32),
          pltpu.VMEM_SHARED((N_PAD, HD), _f32),
          pltpu.HBM((2, N_PAD, HD), _f32),
          pltpu.VMEM_SHARED((N_PAD,), _f32),
          pltpu.SemaphoreType.DMA,
          pltpu.SemaphoreType.DMA,
          pltpu.SemaphoreType.DMA,
      ],
  )
  return f(x_pad, g, ei_p, dist_p, w_vec)


def kernel(x_full, edge_index, edge_dist, w1, w2, w3, w4, w5):
  x = x_full[:N]
  g = x_full[N]
  x_pad = jnp.zeros((N_PAD, D), _f32).at[:N].set(x)
  pad = E_PAD - E
  src_p = jnp.concatenate(
      [edge_index[0], jnp.zeros((pad,), _i32)]).reshape(NS, CPT, 128)
  dst_p = jnp.concatenate(
      [edge_index[1], jnp.full((pad,), N, _i32)]).reshape(NS, CPT, 128)
  ei_p = jnp.stack([src_p, dst_p], axis=2)
  dist_p = jnp.concatenate(
      [edge_dist, jnp.zeros((pad,), _f32)]).reshape(NS, CPT, 128)
  w_vec = jnp.stack([w1, w2, w3, w4, w5] + [jnp.float32(0.0)] * 11)
  out = _run(x_pad, g, ei_p, dist_p, w_vec)
  return out[:N]


# speedup vs baseline: 2.4988x; 2.3039x over previous
import jax
import jax.numpy as jnp
from jax import lax
from jax.experimental import pallas as pl
from jax.experimental.pallas import tpu as pltpu
from jax.experimental.pallas import tpu_sc as plsc

N = 10000
E = 320000
D = 128
P = 5
HD = D // 2
NS = 16
L = 16

N_PAD = 10240
RPT = N_PAD // NS
RQ = RPT // 128

CPT = 160
EPT = CPT * 128
E_PAD = NS * EPT
GRP = 4
NB = 5

_f32 = jnp.float32
_i32 = jnp.int32


def _body(x_hbm, g_hbm, ei_hbm, dist_hbm, w_hbm, out_hbm,
          eix_v, dtmp_v, rows_v, trav_v, g_v, w_v,
          u_sh, acc_sh, base_hbm, travel_sh, semG, semS, semI):
  c = lax.axis_index("c")
  s = lax.axis_index("s")
  row0 = s * RPT
  col0 = c * HD

  pltpu.sync_copy(w_hbm, w_v)
  pltpu.sync_copy(g_hbm.at[pl.ds(col0, HD)], g_v)
  wv = w_v[...]
  w1s, w2s, w3s, w4s, w5s = wv[0], wv[1], wv[2], wv[3], wv[4]
  g4 = [g_v[pl.ds(cc * L, L)] for cc in range(HD // L)]

  @pl.loop(0, 128 // L)
  def _(i):
    trav_v[pl.ds(i * L, L)] = jnp.zeros((L,), _f32)

  for q in range(RQ):
    pltpu.sync_copy(trav_v, travel_sh.at[pl.ds(row0 + 128 * q, 128)])
  plsc.subcore_barrier()

  @pl.loop(0, CPT // GRP)
  def _(g):
    j0 = g * GRP
    gets = [
        pltpu.async_copy(dist_hbm.at[s, j0 + b], dtmp_v.at[b], semG)
        for b in range(GRP)
    ] + [
        pltpu.async_copy(ei_hbm.at[s, j0 + b], eix_v.at[b], semI)
        for b in range(GRP)
    ]
    for h in gets:
      h.wait()
    puts = [
        pltpu.async_copy(dtmp_v.at[b], travel_sh.at[eix_v.at[b, 1]],
                         semS, add=True)
        for b in range(GRP)
    ]
    for h in puts:
      h.wait()
  plsc.subcore_barrier()

  for q in range(RQ):
    r0 = row0 + 128 * q
    x_v = rows_v.at[0]
    b_v = rows_v.at[1]
    pltpu.sync_copy(x_hbm.at[pl.ds(r0, 128), pl.ds(col0, HD)], x_v)
    pltpu.sync_copy(travel_sh.at[pl.ds(r0, 128)], trav_v)

    @pl.loop(0, 128 // L)
    def _(rg):
      tv16 = trav_v[pl.ds(rg * L, L)]
      tvw16 = w4s * jnp.maximum(w5s * tv16, 0.0)
      for i in range(L):
        r = rg * L + i
        tvw = tvw16[i]
        for cc in range(HD // L):
          sl = pl.ds(cc * L, L)
          b_v[r, sl] = w1s * x_v[r, sl] + w2s * g4[cc] + tvw

    pltpu.sync_copy(b_v, base_hbm.at[c, pl.ds(r0, 128)])
    pltpu.sync_copy(x_v, u_sh.at[pl.ds(r0, 128)])

  plsc.subcore_barrier()

  NI = 12

  def fire_i(j):
    pltpu.async_copy(ei_hbm.at[s, j], eix_v.at[j % NI], semI)

  def wait_i(j):
    pltpu.make_async_copy(ei_hbm.at[s, j], eix_v.at[j % NI], semI).wait()

  def fire_g(j):
    pltpu.async_copy(u_sh.at[eix_v.at[j % NI, 0]], rows_v.at[j % NB], semG)

  def wait_g(j):
    pltpu.make_async_copy(u_sh.at[eix_v.at[j % NI, 0]], rows_v.at[j % NB],
                          semG).wait()

  def fire_s(j):
    pltpu.async_copy(rows_v.at[j % NB], acc_sh.at[eix_v.at[j % NI, 1]],
                     semS, add=True)

  def wait_s(j):
    pltpu.make_async_copy(rows_v.at[j % NB], acc_sh.at[eix_v.at[j % NI, 1]],
                          semS).wait()

  def one_round(is_last):
    z_v = rows_v.at[2]

    @pl.loop(0, 128)
    def _(r):
      for cc in range(HD // L):
        z_v[r, pl.ds(cc * L, L)] = jnp.zeros((L,), _f32)

    for q in range(RQ):
      pltpu.sync_copy(z_v, acc_sh.at[pl.ds(row0 + 128 * q, 128)])
    plsc.subcore_barrier()

    for j in range(8):
      fire_i(j)
    for j in range(2):
      wait_i(j)
      fire_g(j)
    for j in range(3):
      wait_g(j)
      fire_s(j)
      fire_i(j + 8)
      wait_i(j + 2)
      fire_g(j + 2)

    def steady(j):
      wait_g(j)
      fire_s(j)
      wait_s(j - 3)
      fire_i(j + 8)
      wait_i(j + 2)
      fire_g(j + 2)

    @pl.loop(3, 152)
    def _(j):
      steady(j)

    for j in range(152, 158):
      wait_g(j)
      fire_s(j)
      wait_s(j - 3)
      wait_i(j + 2)
      fire_g(j + 2)
    for j in range(158, 160):
      wait_g(j)
      fire_s(j)
      wait_s(j - 3)
    for j in range(157, 160):
      wait_s(j)

    plsc.subcore_barrier()

    for q in range(RQ):
      r0 = row0 + 128 * q
      a_v = rows_v.at[0]
      b_v = rows_v.at[1]
      pltpu.sync_copy(acc_sh.at[pl.ds(r0, 128)], a_v)
      pltpu.sync_copy(base_hbm.at[c, pl.ds(r0, 128)], b_v)

      @pl.loop(0, 128)
      def _(r):
        for cc in range(HD // L):
          sl = pl.ds(cc * L, L)
          val = b_v[r, sl] + w3s * a_v[r, sl]
          if is_last:
            val = jnp.maximum(val, 0.0)
          a_v[r, sl] = val

      if is_last:
        pltpu.sync_copy(a_v, out_hbm.at[pl.ds(r0, 128), pl.ds(col0, HD)])
      else:
        pltpu.sync_copy(a_v, u_sh.at[pl.ds(r0, 128)])

    if not is_last:
      plsc.subcore_barrier()

  @pl.loop(0, P - 1)
  def _(k):
    one_round(False)

  one_round(True)


@jax.jit
def _run(x_pad, g, ei_p, dist_p, w_vec):
  mesh = plsc.VectorSubcoreMesh(core_axis_name="c", subcore_axis_name="s")
  f = pl.kernel(
      _body,
      out_type=jax.ShapeDtypeStruct((N_PAD, D), _f32),
      mesh=mesh,
      compiler_params=pltpu.CompilerParams(use_tc_tiling_on_sc=False),
      scratch_types=[
          pltpu.VMEM((12, 2, 128), _i32),
          pltpu.VMEM((GRP, 128), _f32),
          pltpu.VMEM((NB, 128, HD), _f32),
          pltpu.VMEM((128,), _f32),
          pltpu.VMEM((HD,), _f32),
          pltpu.VMEM((16,), _f32),
          pltpu.VMEM_SHARED((N_PAD, HD), _f32),
          pltpu.VMEM_SHARED((N_PAD, HD), _f32),
          pltpu.HBM((2, N_PAD, HD), _f32),
          pltpu.VMEM_SHARED((N_PAD,), _f32),
          pltpu.SemaphoreType.DMA,
          pltpu.SemaphoreType.DMA,
          pltpu.SemaphoreType.DMA,
      ],
  )
  return f(x_pad, g, ei_p, dist_p, w_vec)


def kernel(x_full, edge_index, edge_dist, w1, w2, w3, w4, w5):
  x = x_full[:N]
  g = x_full[N]
  x_pad = jnp.zeros((N_PAD, D), _f32).at[:N].set(x)
  pad = E_PAD - E
  src_p = jnp.concatenate(
      [edge_index[0], jnp.zeros((pad,), _i32)]).reshape(NS, CPT, 128)
  dst_p = jnp.concatenate(
      [edge_index[1], jnp.full((pad,), N, _i32)]).reshape(NS, CPT, 128)
  ei_p = jnp.stack([src_p, dst_p], axis=2)
  dist_p = jnp.concatenate(
      [edge_dist, jnp.zeros((pad,), _f32)]).reshape(NS, CPT, 128)
  w_vec = jnp.stack([w1, w2, w3, w4, w5] + [jnp.float32(0.0)] * 11)
  out = _run(x_pad, g, ei_p, dist_p, w_vec)
  return out[:N]


# speedup vs baseline: 2.5546x; 1.0223x over previous
import jax
import jax.numpy as jnp
from jax import lax
from jax.experimental import pallas as pl
from jax.experimental.pallas import tpu as pltpu
from jax.experimental.pallas import tpu_sc as plsc

N = 10000
E = 320000
D = 128
P = 5
HD = D // 2
NS = 16
L = 16

N_PAD = 10240
RPT = N_PAD // NS
RQ = RPT // 128

CPT = 160
EPT = CPT * 128
E_PAD = NS * EPT
GRP = 4
NB = 5

_f32 = jnp.float32
_i32 = jnp.int32


def _body(x_hbm, g_hbm, ei_hbm, dist_hbm, w_hbm, out_hbm,
          eix_v, dtmp_v, rows_v, trav_v, g_v, w_v,
          u_sh, acc_sh, base_hbm, travel_sh, semG, semS, semI):
  c = lax.axis_index("c")
  s = lax.axis_index("s")
  row0 = s * RPT
  col0 = c * HD

  pltpu.sync_copy(w_hbm, w_v)
  pltpu.sync_copy(g_hbm.at[pl.ds(col0, HD)], g_v)
  wv = w_v[...]
  w1s, w2s, w3s, w4s, w5s = wv[0], wv[1], wv[2], wv[3], wv[4]
  g4 = [g_v[pl.ds(cc * L, L)] for cc in range(HD // L)]

  @pl.loop(0, 128 // L)
  def _(i):
    trav_v[pl.ds(i * L, L)] = jnp.zeros((L,), _f32)

  for q in range(RQ):
    pltpu.sync_copy(trav_v, travel_sh.at[pl.ds(row0 + 128 * q, 128)])
  plsc.subcore_barrier()

  @pl.loop(0, CPT // GRP)
  def _(g):
    j0 = g * GRP
    gets = [
        pltpu.async_copy(dist_hbm.at[s, j0 + b], dtmp_v.at[b], semG)
        for b in range(GRP)
    ] + [
        pltpu.async_copy(ei_hbm.at[s, j0 + b], eix_v.at[b], semI)
        for b in range(GRP)
    ]
    for h in gets:
      h.wait()
    puts = [
        pltpu.async_copy(dtmp_v.at[b], travel_sh.at[eix_v.at[b, 1]],
                         semS, add=True)
        for b in range(GRP)
    ]
    for h in puts:
      h.wait()
  plsc.subcore_barrier()

  z_v = rows_v.at[2]

  @pl.loop(0, 128)
  def _(r):
    for cc in range(HD // L):
      z_v[r, pl.ds(cc * L, L)] = jnp.zeros((L,), _f32)

  for q in range(RQ):
    r0 = row0 + 128 * q
    x_v = rows_v.at[0]
    b_v = rows_v.at[1]
    pltpu.sync_copy(z_v, acc_sh.at[pl.ds(r0, 128)])
    pltpu.sync_copy(x_hbm.at[pl.ds(r0, 128), pl.ds(col0, HD)], x_v)
    pltpu.sync_copy(travel_sh.at[pl.ds(r0, 128)], trav_v)

    @pl.loop(0, 128 // L)
    def _(rg):
      tv16 = trav_v[pl.ds(rg * L, L)]
      tvw16 = w4s * jnp.maximum(w5s * tv16, 0.0)
      for i in range(L):
        r = rg * L + i
        tvw = tvw16[i]
        for cc in range(HD // L):
          sl = pl.ds(cc * L, L)
          b_v[r, sl] = w1s * x_v[r, sl] + w2s * g4[cc] + tvw

    pltpu.sync_copy(b_v, base_hbm.at[c, pl.ds(r0, 128)])
    pltpu.sync_copy(x_v, u_sh.at[pl.ds(r0, 128)])

  plsc.subcore_barrier()

  NI = 12

  def fire_i(j):
    pltpu.async_copy(ei_hbm.at[s, j], eix_v.at[j % NI], semI)

  def wait_i(j):
    pltpu.make_async_copy(ei_hbm.at[s, j], eix_v.at[j % NI], semI).wait()

  def fire_g(j):
    pltpu.async_copy(u_sh.at[eix_v.at[j % NI, 0]], rows_v.at[j % NB], semG)

  def wait_g(j):
    pltpu.make_async_copy(u_sh.at[eix_v.at[j % NI, 0]], rows_v.at[j % NB],
                          semG).wait()

  def fire_s(j):
    pltpu.async_copy(rows_v.at[j % NB], acc_sh.at[eix_v.at[j % NI, 1]],
                     semS, add=True)

  def wait_s(j):
    pltpu.make_async_copy(rows_v.at[j % NB], acc_sh.at[eix_v.at[j % NI, 1]],
                          semS).wait()

  def one_round(is_last):
    for j in range(8):
      fire_i(j)
    for j in range(2):
      wait_i(j)
      fire_g(j)
    for j in range(3):
      wait_g(j)
      fire_s(j)
      fire_i(j + 8)
      wait_i(j + 2)
      fire_g(j + 2)

    def steady(j):
      wait_g(j)
      fire_s(j)
      wait_s(j - 3)
      fire_i(j + 8)
      wait_i(j + 2)
      fire_g(j + 2)

    @pl.loop(3, 149)
    def _(j):
      steady(j)

    for j in range(149, 155):
      wait_g(j)
      fire_s(j)
      wait_s(j - 3)
      wait_i(j + 2)
      fire_g(j + 2)
    for j in range(155, 157):
      wait_g(j)
      fire_s(j)
      wait_s(j - 3)
    for j in range(154, 157):
      wait_s(j)

    plsc.subcore_barrier()

    if not is_last:
      z_v = rows_v.at[2]

      @pl.loop(0, 128)
      def _(r):
        for cc in range(HD // L):
          z_v[r, pl.ds(cc * L, L)] = jnp.zeros((L,), _f32)

    for q in range(RQ):
      r0 = row0 + 128 * q
      a_v = rows_v.at[0]
      b_v = rows_v.at[1]
      pltpu.sync_copy(acc_sh.at[pl.ds(r0, 128)], a_v)
      if not is_last:
        pltpu.sync_copy(rows_v.at[2], acc_sh.at[pl.ds(r0, 128)])
      pltpu.sync_copy(base_hbm.at[c, pl.ds(r0, 128)], b_v)

      @pl.loop(0, 128)
      def _(r):
        for cc in range(HD // L):
          sl = pl.ds(cc * L, L)
          val = b_v[r, sl] + w3s * a_v[r, sl]
          if is_last:
            val = jnp.maximum(val, 0.0)
          a_v[r, sl] = val

      if is_last:
        pltpu.sync_copy(a_v, out_hbm.at[pl.ds(r0, 128), pl.ds(col0, HD)])
      else:
        pltpu.sync_copy(a_v, u_sh.at[pl.ds(r0, 128)])

    if not is_last:
      plsc.subcore_barrier()

  @pl.loop(0, P - 1)
  def _(k):
    one_round(False)

  one_round(True)


@jax.jit
def _run(x_pad, g, ei_p, dist_p, w_vec):
  mesh = plsc.VectorSubcoreMesh(core_axis_name="c", subcore_axis_name="s")
  f = pl.kernel(
      _body,
      out_type=jax.ShapeDtypeStruct((N_PAD, D), _f32),
      mesh=mesh,
      compiler_params=pltpu.CompilerParams(use_tc_tiling_on_sc=False),
      scratch_types=[
          pltpu.VMEM((12, 2, 128), _i32),
          pltpu.VMEM((GRP, 128), _f32),
          pltpu.VMEM((NB, 128, HD), _f32),
          pltpu.VMEM((128,), _f32),
          pltpu.VMEM((HD,), _f32),
          pltpu.VMEM((16,), _f32),
          pltpu.VMEM_SHARED((N_PAD, HD), _f32),
          pltpu.VMEM_SHARED((N_PAD, HD), _f32),
          pltpu.HBM((2, N_PAD, HD), _f32),
          pltpu.VMEM_SHARED((N_PAD,), _f32),
          pltpu.SemaphoreType.DMA,
          pltpu.SemaphoreType.DMA,
          pltpu.SemaphoreType.DMA,
      ],
  )
  return f(x_pad, g, ei_p, dist_p, w_vec)


def kernel(x_full, edge_index, edge_dist, w1, w2, w3, w4, w5):
  x = x_full[:N]
  g = x_full[N]
  x_pad = jnp.zeros((N_PAD, D), _f32).at[:N].set(x)
  ppt = EPT - E // NS
  src_p = jnp.pad(edge_index[0].reshape(NS, E // NS),
                  ((0, 0), (0, ppt))).reshape(NS, CPT, 128)
  dst_p = jnp.pad(edge_index[1].reshape(NS, E // NS), ((0, 0), (0, ppt)),
                  constant_values=N).reshape(NS, CPT, 128)
  ei_p = jnp.stack([src_p, dst_p], axis=2)
  dist_p = jnp.pad(edge_dist.reshape(NS, E // NS),
                   ((0, 0), (0, ppt))).reshape(NS, CPT, 128)
  w_vec = jnp.stack([w1, w2, w3, w4, w5] + [jnp.float32(0.0)] * 11)
  out = _run(x_pad, g, ei_p, dist_p, w_vec)
  return out[:N]
